# baseline (speedup 1.0000x reference)
import functools

import jax
import jax.numpy as jnp
from jax import lax
from jax.experimental import pallas as pl
from jax.experimental.pallas import tpu as pltpu
from jax.experimental.pallas import tpu_sc as plsc

N = 10000
D = 128
E = 320000
NC, NS = 2, 16
NW = NC * NS
CB = 128
CH = 78
NT = E // CB - CH * NW
ND = 10240
RPTD = ND // NS
TAILR = N - (NS - 1) * RPTD
IR = 4


def _mesh():
    return plsc.VectorSubcoreMesh(core_axis_name="c", subcore_axis_name="s")


def _make_deg():
    @functools.partial(
        pl.kernel,
        out_type=jax.ShapeDtypeStruct((NC, ND), jnp.float32),
        mesh=_mesh(),
        scratch_types=[
            pltpu.VMEM((2, CB), jnp.int32),
            pltpu.VMEM((CB,), jnp.float32),
            pltpu.VMEM((RPTD,), jnp.float32),
            pltpu.VMEM_SHARED((ND,), jnp.float32),
            pltpu.SemaphoreType.DMA,
            pltpu.SemaphoreType.DMA,
        ],
    )
    def deg(ei, out, idx_v, ones_v, zvec, accd, isem, ssem):
        c = lax.axis_index("c")
        s = lax.axis_index("s")
        wid = c * NS + s

        def fill_ones(k, carry):
            ones_v[pl.ds(k * 16, 16)] = jnp.full((16,), 1.0, jnp.float32)
            return carry

        lax.fori_loop(0, CB // 16, fill_ones, 0)

        def fill_zero(k, carry):
            zvec[pl.ds(k * 16, 16)] = jnp.zeros((16,), jnp.float32)
            return carry

        lax.fori_loop(0, RPTD // 16, fill_zero, 0)
        pltpu.sync_copy(zvec, accd.at[pl.ds(s * RPTD, RPTD)])
        plsc.subcore_barrier()

        def idx_dma(j, b):
            return pltpu.make_async_copy(
                ei.at[1, pl.ds((j * NW + wid) * CB, CB)], idx_v.at[b], isem)

        idx_dma(0, 0).start()

        def chunk(j, carry):
            for b in range(2):
                jj = 2 * j + b
                idx_dma(jj, b).wait()

                @pl.when(jj + 1 < CH)
                def _next():
                    idx_dma(jj + 1, 1 - b).start()

                pltpu.async_copy(ones_v, accd.at[idx_v.at[b]], ssem,
                                 add=True).wait()
            return carry

        lax.fori_loop(0, CH // 2, chunk, 0)

        @pl.when(wid < NT)
        def _tail():
            pltpu.sync_copy(ei.at[1, pl.ds((CH * NW + wid) * CB, CB)],
                            idx_v.at[0])
            pltpu.async_copy(ones_v, accd.at[idx_v.at[0]], ssem,
                             add=True).wait()

        plsc.subcore_barrier()

        pltpu.sync_copy(accd.at[pl.ds(s * RPTD, RPTD)],
                        out.at[c, pl.ds(s * RPTD, RPTD)])

    return deg


def _make_agg():
    @functools.partial(
        pl.kernel,
        out_type=jax.ShapeDtypeStruct((NC, N, D), jnp.float32),
        mesh=_mesh(),
        scratch_types=[
            pltpu.VMEM((IR, 2, CB), jnp.int32),
            pltpu.VMEM((2, CB, D), jnp.float32),
            pltpu.VMEM_SHARED((ND, D), jnp.float32),
            pltpu.SemaphoreType.DMA,
            pltpu.SemaphoreType.DMA,
            pltpu.SemaphoreType.DMA,
            pltpu.SemaphoreType.DMA,
        ],
    )
    def agg(g_hbm, ei, out, idx_v, rows_v, acc, isem, gsem,
            ssem0, ssem1):
        c = lax.axis_index("c")
        s = lax.axis_index("s")
        wid = c * NS + s

        def fill_zero(k, carry):
            rows_v[0, k // 8, pl.ds((k % 8) * 16, 16)] = jnp.zeros(
                (16,), jnp.float32)
            return carry

        lax.fori_loop(0, CB * (D // 16), fill_zero, 0)

        def zero_acc(j, carry):
            pltpu.sync_copy(rows_v.at[0], acc.at[pl.ds(s * RPTD + j * CB, CB)])
            return carry

        lax.fori_loop(0, RPTD // CB, zero_acc, 0)
        plsc.subcore_barrier()

        ssems = (ssem0, ssem1)

        def idx_dma(j, d):
            return pltpu.make_async_copy(
                ei.at[d, pl.ds((j * NW + wid) * CB, CB)],
                idx_v.at[lax.rem(j, IR), d], isem)

        def idx_start(j):
            idx_dma(j, 0).start()
            idx_dma(j, 1).start()

        def idx_wait(j):
            idx_dma(j, 0).wait()
            idx_dma(j, 1).wait()

        def gather(j, b):
            return pltpu.make_async_copy(
                g_hbm.at[idx_v.at[lax.rem(j, IR), 0]], rows_v.at[b], gsem)

        def scatter(j, b):
            return pltpu.make_async_copy(
                rows_v.at[b], acc.at[idx_v.at[lax.rem(j, IR), 1]], ssems[b])

        idx_start(0)
        idx_start(1)
        idx_start(2)
        idx_wait(0)
        pltpu.async_copy(g_hbm.at[idx_v.at[0, 0]], rows_v.at[0], gsem)

        def pipe(i, carry):
            for b in range(2):
                j = 2 * i + b
                gather(j, b).wait()
                pltpu.async_copy(rows_v.at[b],
                                 acc.at[idx_v.at[lax.rem(j, IR), 1]],
                                 ssems[b], add=True)

                @pl.when(j >= 1)
                def _drain_other():
                    scatter(j - 1, 1 - b).wait()

                @pl.when(j + 1 < CH)
                def _next_gather():
                    idx_wait(j + 1)
                    gather(j + 1, 1 - b).start()

                @pl.when(j + 3 < CH)
                def _next_idx():
                    idx_start(j + 3)
            return carry

        lax.fori_loop(0, CH // 2, pipe, 0)
        scatter(CH - 1, 1).wait()

        @pl.when(wid < NT)
        def _tail():
            base = (CH * NW + wid) * CB
            pltpu.sync_copy(ei.at[0, pl.ds(base, CB)], idx_v.at[0, 0])
            pltpu.sync_copy(ei.at[1, pl.ds(base, CB)], idx_v.at[0, 1])
            pltpu.async_copy(g_hbm.at[idx_v.at[0, 0]], rows_v.at[0],
                             gsem).wait()
            pltpu.async_copy(rows_v.at[0], acc.at[idx_v.at[0, 1]],
                             ssem0, add=True).wait()

        plsc.subcore_barrier()

        @pl.when(s < NS - 1)
        def _dump_full():
            pltpu.sync_copy(acc.at[pl.ds(s * RPTD, RPTD)],
                            out.at[c, pl.ds(s * RPTD, RPTD)])

        @pl.when(s == NS - 1)
        def _dump_tail():
            pltpu.sync_copy(acc.at[pl.ds((NS - 1) * RPTD, TAILR)],
                            out.at[c, pl.ds((NS - 1) * RPTD, TAILR)])

    return agg


_R = 1000


def _dinv(d0, d1):
    return lax.rsqrt(jnp.maximum(d0 + d1 + 1.0, 1.0))


def _g1_body(x_ref, w_ref, d0_ref, d1_ref, o_ref):
    h = jnp.dot(x_ref[...], w_ref[...], preferred_element_type=jnp.float32)
    o_ref[...] = h * _dinv(d0_ref[...], d1_ref[...])


def _mm_scale(xp, w, d0, d1):
    return pl.pallas_call(
        _g1_body,
        grid=(N // _R,),
        in_specs=[
            pl.BlockSpec((_R, D), lambda i: (i, 0)),
            pl.BlockSpec((D, D), lambda i: (0, 0)),
            pl.BlockSpec((_R, 1), lambda i: (i, 0)),
            pl.BlockSpec((_R, 1), lambda i: (i, 0)),
        ],
        out_specs=pl.BlockSpec((_R, D), lambda i: (i, 0)),
        out_shape=jax.ShapeDtypeStruct((N, D), jnp.float32),
    )(xp, w, d0, d1)


def _mid_body(z0_ref, z1_ref, g1_ref, d0_ref, d1_ref, b1_ref, a1_ref, w2_ref,
              o_ref):
    dinv = _dinv(d0_ref[...], d1_ref[...])
    z = z0_ref[...] + z1_ref[...] + g1_ref[...]
    out1 = z * dinv + b1_ref[...]
    h = jnp.where(out1 >= 0.0, out1, a1_ref[...] * out1)
    o_ref[...] = jnp.dot(h, w2_ref[...],
                         preferred_element_type=jnp.float32) * dinv


def _mid(zf, g1, df, b1, a1, w2):
    nb = N // _R
    return pl.pallas_call(
        _mid_body,
        grid=(nb,),
        in_specs=[
            pl.BlockSpec((_R, D), lambda i: (i, 0)),
            pl.BlockSpec((_R, D), lambda i: (i + nb, 0)),
            pl.BlockSpec((_R, D), lambda i: (i, 0)),
            pl.BlockSpec((_R, 1), lambda i: (i, 0)),
            pl.BlockSpec((_R, 1), lambda i: (i + nb, 0)),
            pl.BlockSpec((1, D), lambda i: (0, 0)),
            pl.BlockSpec((1, D), lambda i: (0, 0)),
            pl.BlockSpec((D, D), lambda i: (0, 0)),
        ],
        out_specs=pl.BlockSpec((_R, D), lambda i: (i, 0)),
        out_shape=jax.ShapeDtypeStruct((N, D), jnp.float32),
    )(zf, zf, g1, df, df, b1, a1, w2)


def _final_body(z0_ref, z1_ref, g2_ref, d0_ref, d1_ref, b2_ref, o_ref):
    dinv = _dinv(d0_ref[...], d1_ref[...])
    z = z0_ref[...] + z1_ref[...] + g2_ref[...]
    o_ref[...] = z * dinv + b2_ref[...]


def _final(zf, g2, df, b2):
    nb = N // _R
    return pl.pallas_call(
        _final_body,
        grid=(nb,),
        in_specs=[
            pl.BlockSpec((_R, D), lambda i: (i, 0)),
            pl.BlockSpec((_R, D), lambda i: (i + nb, 0)),
            pl.BlockSpec((_R, D), lambda i: (i, 0)),
            pl.BlockSpec((_R, 1), lambda i: (i, 0)),
            pl.BlockSpec((_R, 1), lambda i: (i + nb, 0)),
            pl.BlockSpec((1, D), lambda i: (0, 0)),
        ],
        out_specs=pl.BlockSpec((_R, D), lambda i: (i, 0)),
        out_shape=jax.ShapeDtypeStruct((N, D), jnp.float32),
    )(zf, zf, g2, df, df, b2)


def kernel(x, edge_index, W1, b1, a1, W2, b2):
    ei = edge_index.astype(jnp.int32)

    degs = _make_deg()(ei)
    df = degs[:, :N].reshape(NC * N, 1)
    g1 = _mm_scale(x, W1, df[:N], df[N:])

    agg = _make_agg()
    z1 = agg(g1, ei).reshape(NC * N, D)
    g2 = _mid(z1, g1, df, b1.reshape(1, D), a1.reshape(1, D), W2)
    z2 = agg(g2, ei).reshape(NC * N, D)
    return _final(z2, g2, df, b2.reshape(1, D))


# speedup vs baseline: 1.0794x; 1.0794x over previous
import functools

import jax
import jax.numpy as jnp
from jax import lax
from jax.experimental import pallas as pl
from jax.experimental.pallas import tpu as pltpu
from jax.experimental.pallas import tpu_sc as plsc

N = 10000
D = 128
E = 320000
NC, NS = 2, 16
NW = NC * NS
CB = 125
CH = 80
ND = 10240
RPTD = ND // NS
TAILR = N - (NS - 1) * RPTD
IR = 4


def _mesh():
    return plsc.VectorSubcoreMesh(core_axis_name="c", subcore_axis_name="s")


def _make_deg():
    @functools.partial(
        pl.kernel,
        out_type=jax.ShapeDtypeStruct((NC, ND), jnp.float32),
        mesh=_mesh(),
        scratch_types=[
            pltpu.VMEM((CH, CB), jnp.int32),
            pltpu.VMEM((128,), jnp.float32),
            pltpu.VMEM((RPTD,), jnp.float32),
            pltpu.VMEM_SHARED((ND,), jnp.float32),
            pltpu.SemaphoreType.DMA,
        ],
    )
    def deg(sdw, out, idx_v, ones_v, zvec, accd, ssem):
        c = lax.axis_index("c")
        s = lax.axis_index("s")
        wid = c * NS + s

        def fill_ones(k, carry):
            ones_v[pl.ds(k * 16, 16)] = jnp.full((16,), 1.0, jnp.float32)
            return carry

        lax.fori_loop(0, 8, fill_ones, 0)

        def fill_zero(k, carry):
            zvec[pl.ds(k * 16, 16)] = jnp.zeros((16,), jnp.float32)
            return carry

        lax.fori_loop(0, RPTD // 16, fill_zero, 0)
        pltpu.sync_copy(zvec, accd.at[pl.ds(s * RPTD, RPTD)])
        pltpu.sync_copy(sdw.at[1, wid], idx_v)
        plsc.subcore_barrier()

        def chunk(j, carry):
            pltpu.async_copy(ones_v.at[pl.ds(0, CB)], accd.at[idx_v.at[j]],
                             ssem, add=True).wait()
            return carry

        lax.fori_loop(0, CH, chunk, 0)
        plsc.subcore_barrier()

        pltpu.sync_copy(accd.at[pl.ds(s * RPTD, RPTD)],
                        out.at[c, pl.ds(s * RPTD, RPTD)])

    return deg


def _make_agg():
    @functools.partial(
        pl.kernel,
        out_type=jax.ShapeDtypeStruct((NC, N, D), jnp.float32),
        mesh=_mesh(),
        scratch_types=[
            pltpu.VMEM((IR, 2, CB), jnp.int32),
            pltpu.VMEM((2, CB, D), jnp.float32),
            pltpu.VMEM_SHARED((ND, D), jnp.float32),
            pltpu.SemaphoreType.DMA,
            pltpu.SemaphoreType.DMA,
            pltpu.SemaphoreType.DMA,
            pltpu.SemaphoreType.DMA,
        ],
    )
    def agg(g_hbm, sdw, out, idx_v, rows_v, acc, isem, gsem,
            ssem0, ssem1):
        c = lax.axis_index("c")
        s = lax.axis_index("s")
        wid = c * NS + s

        def fill_zero(k, carry):
            rows_v[0, k // 8, pl.ds((k % 8) * 16, 16)] = jnp.zeros(
                (16,), jnp.float32)
            return carry

        lax.fori_loop(0, CB * (D // 16), fill_zero, 0)

        def zero_acc(j, carry):
            pltpu.sync_copy(rows_v.at[0, pl.ds(0, 120)],
                            acc.at[pl.ds(s * RPTD + j * 120, 120)])
            return carry

        lax.fori_loop(0, 5, zero_acc, 0)
        pltpu.sync_copy(rows_v.at[0, pl.ds(0, 40)],
                        acc.at[pl.ds(s * RPTD + 600, 40)])
        plsc.subcore_barrier()

        ssems = (ssem0, ssem1)

        def idx_dma(j, d):
            return pltpu.make_async_copy(sdw.at[d, wid, j],
                                         idx_v.at[lax.rem(j, IR), d], isem)

        def idx_start(j):
            idx_dma(j, 0).start()
            idx_dma(j, 1).start()

        def idx_wait(j):
            idx_dma(j, 0).wait()
            idx_dma(j, 1).wait()

        def gather(j, b):
            return pltpu.make_async_copy(
                g_hbm.at[idx_v.at[lax.rem(j, IR), 0]], rows_v.at[b], gsem)

        def scatter(j, b):
            return pltpu.make_async_copy(
                rows_v.at[b], acc.at[idx_v.at[lax.rem(j, IR), 1]], ssems[b])

        idx_start(0)
        idx_start(1)
        idx_start(2)
        idx_wait(0)
        pltpu.async_copy(g_hbm.at[idx_v.at[0, 0]], rows_v.at[0], gsem)

        def pipe(i, carry):
            for b in range(2):
                j = 2 * i + b
                gather(j, b).wait()
                pltpu.async_copy(rows_v.at[b],
                                 acc.at[idx_v.at[lax.rem(j, IR), 1]],
                                 ssems[b], add=True)

                @pl.when(j >= 1)
                def _drain_other():
                    scatter(j - 1, 1 - b).wait()

                @pl.when(j + 1 < CH)
                def _next_gather():
                    idx_wait(j + 1)
                    gather(j + 1, 1 - b).start()

                @pl.when(j + 3 < CH)
                def _next_idx():
                    idx_start(j + 3)
            return carry

        lax.fori_loop(0, CH // 2, pipe, 0)
        scatter(CH - 1, 1).wait()
        plsc.subcore_barrier()

        @pl.when(s < NS - 1)
        def _dump_full():
            pltpu.sync_copy(acc.at[pl.ds(s * RPTD, RPTD)],
                            out.at[c, pl.ds(s * RPTD, RPTD)])

        @pl.when(s == NS - 1)
        def _dump_tail():
            pltpu.sync_copy(acc.at[pl.ds((NS - 1) * RPTD, TAILR)],
                            out.at[c, pl.ds((NS - 1) * RPTD, TAILR)])

    return agg


_R = 1000


def _dinv(d0, d1):
    return lax.rsqrt(jnp.maximum(d0 + d1 + 1.0, 1.0))


def _g1_body(x_ref, w_ref, d0_ref, d1_ref, o_ref):
    h = jnp.dot(x_ref[...], w_ref[...], preferred_element_type=jnp.float32)
    o_ref[...] = h * _dinv(d0_ref[...], d1_ref[...])


def _mm_scale(xp, w, d0, d1):
    return pl.pallas_call(
        _g1_body,
        grid=(N // _R,),
        in_specs=[
            pl.BlockSpec((_R, D), lambda i: (i, 0)),
            pl.BlockSpec((D, D), lambda i: (0, 0)),
            pl.BlockSpec((_R, 1), lambda i: (i, 0)),
            pl.BlockSpec((_R, 1), lambda i: (i, 0)),
        ],
        out_specs=pl.BlockSpec((_R, D), lambda i: (i, 0)),
        out_shape=jax.ShapeDtypeStruct((N, D), jnp.float32),
    )(xp, w, d0, d1)


def _mid_body(z0_ref, z1_ref, g1_ref, d0_ref, d1_ref, b1_ref, a1_ref, w2_ref,
              o_ref):
    dinv = _dinv(d0_ref[...], d1_ref[...])
    z = z0_ref[...] + z1_ref[...] + g1_ref[...]
    out1 = z * dinv + b1_ref[...]
    h = jnp.where(out1 >= 0.0, out1, a1_ref[...] * out1)
    o_ref[...] = jnp.dot(h, w2_ref[...],
                         preferred_element_type=jnp.float32) * dinv


def _mid(zf, g1, df, b1, a1, w2):
    nb = N // _R
    return pl.pallas_call(
        _mid_body,
        grid=(nb,),
        in_specs=[
            pl.BlockSpec((_R, D), lambda i: (i, 0)),
            pl.BlockSpec((_R, D), lambda i: (i + nb, 0)),
            pl.BlockSpec((_R, D), lambda i: (i, 0)),
            pl.BlockSpec((_R, 1), lambda i: (i, 0)),
            pl.BlockSpec((_R, 1), lambda i: (i + nb, 0)),
            pl.BlockSpec((1, D), lambda i: (0, 0)),
            pl.BlockSpec((1, D), lambda i: (0, 0)),
            pl.BlockSpec((D, D), lambda i: (0, 0)),
        ],
        out_specs=pl.BlockSpec((_R, D), lambda i: (i, 0)),
        out_shape=jax.ShapeDtypeStruct((N, D), jnp.float32),
    )(zf, zf, g1, df, df, b1, a1, w2)


def _final_body(z0_ref, z1_ref, g2_ref, d0_ref, d1_ref, b2_ref, o_ref):
    dinv = _dinv(d0_ref[...], d1_ref[...])
    z = z0_ref[...] + z1_ref[...] + g2_ref[...]
    o_ref[...] = z * dinv + b2_ref[...]


def _final(zf, g2, df, b2):
    nb = N // _R
    return pl.pallas_call(
        _final_body,
        grid=(nb,),
        in_specs=[
            pl.BlockSpec((_R, D), lambda i: (i, 0)),
            pl.BlockSpec((_R, D), lambda i: (i + nb, 0)),
            pl.BlockSpec((_R, D), lambda i: (i, 0)),
            pl.BlockSpec((_R, 1), lambda i: (i, 0)),
            pl.BlockSpec((_R, 1), lambda i: (i + nb, 0)),
            pl.BlockSpec((1, D), lambda i: (0, 0)),
        ],
        out_specs=pl.BlockSpec((_R, D), lambda i: (i, 0)),
        out_shape=jax.ShapeDtypeStruct((N, D), jnp.float32),
    )(zf, zf, g2, df, df, b2)


def kernel(x, edge_index, W1, b1, a1, W2, b2):
    sdw = edge_index.astype(jnp.int32).reshape(2, NW, CH, CB)

    degs = _make_deg()(sdw)
    df = degs[:, :N].reshape(NC * N, 1)
    g1 = _mm_scale(x, W1, df[:N], df[N:])

    agg = _make_agg()
    z1 = agg(g1, sdw).reshape(NC * N, D)
    g2 = _mid(z1, g1, df, b1.reshape(1, D), a1.reshape(1, D), W2)
    z2 = agg(g2, sdw).reshape(NC * N, D)
    return _final(z2, g2, df, b2.reshape(1, D))
